# baseline (speedup 1.0000x reference)
import jax
import jax.numpy as jnp
from jax.experimental import pallas as pl


def _body(u_ref, i_ref, o_ref):
    o_ref[...] = jnp.sum(u_ref[...] * i_ref[...], axis=0)


def kernel(gu, gi):
    gu = jnp.squeeze(gu)
    gi = jnp.squeeze(gi)
    n, k = gu.shape
    ut = gu.T
    it = gi.T
    bn = 98304
    grid = pl.cdiv(n, bn)
    return pl.pallas_call(
        _body,
        grid=(grid,),
        in_specs=[
            pl.BlockSpec((k, bn), lambda i: (0, i)),
            pl.BlockSpec((k, bn), lambda i: (0, i)),
        ],
        out_specs=pl.BlockSpec((bn,), lambda i: (i,)),
        out_shape=jax.ShapeDtypeStruct((n,), jnp.float32),
    )(ut, it)


# speedup vs baseline: 1.0382x; 1.0382x over previous
import jax
import jax.numpy as jnp
from jax.experimental import pallas as pl


def _body(u_ref, i_ref, o_ref):
    o_ref[...] = jnp.sum(u_ref[...] * i_ref[...], axis=0)


def kernel(gu, gi):
    gu = jnp.squeeze(gu)
    gi = jnp.squeeze(gi)
    n, k = gu.shape
    ut = gu.T
    it = gi.T
    bn = 32768
    grid = pl.cdiv(n, bn)
    return pl.pallas_call(
        _body,
        grid=(grid,),
        in_specs=[
            pl.BlockSpec((k, bn), lambda i: (0, i)),
            pl.BlockSpec((k, bn), lambda i: (0, i)),
        ],
        out_specs=pl.BlockSpec((bn,), lambda i: (i,)),
        out_shape=jax.ShapeDtypeStruct((n,), jnp.float32),
    )(ut, it)
